# baseline (speedup 1.0000x reference)
import functools

import jax
import jax.numpy as jnp
from jax import lax
from jax.experimental import pallas as pl
from jax.experimental.pallas import tpu as pltpu
from jax.experimental.pallas import tpu_sc as plsc

M, J, I, L, W = 16, 2, 3, 2, 9

_B_G, _B_T, _B_H, _B_W, _B_B = W * L, W * L + J * L, W * L + 2 * J * L, \
    W * L + 2 * J * L + J * I * L, W * L + 2 * J * L + J * I * L + L * I
_ROWS = _B_B + L


def _body(a_hbm, out_hbm, a_v, o_v):
    @pl.when((lax.axis_index("c") == 0) & (lax.axis_index("s") == 0))
    def _():
        pltpu.sync_copy(a_hbm, a_v)

        def row(r):
            return a_v[r]

        cap = [jnp.zeros((16,), jnp.float32) for _ in range(I)]
        for j in range(J):
            gam = [row(_B_G + j * L + l) for l in range(L)]
            sig = [1.0 / (1.0 + jnp.exp(-10.0 * (g - 0.5))) for g in gam]
            tem = [row(_B_T + j * L + l) for l in range(L)]
            best_q = None
            sel = [None] * L
            for w in range(W):
                d0 = tem[0] - row(w * L + 0)
                q = sig[0] * d0 * d0
                for l in range(1, L):
                    dl = tem[l] - row(w * L + l)
                    q = q + sig[l] * dl * dl
                if best_q is None:
                    best_q = q
                    sel = [row(w * L + l) for l in range(L)]
                else:
                    take = q < best_q
                    best_q = jnp.where(take, q, best_q)
                    sel = [jnp.where(take, row(w * L + l), sel[l])
                           for l in range(L)]
            gs = [jnp.where(g > 0.5, sel[l], 0.0)
                  for l, g in enumerate(gam)]
            for i in range(I):
                for l in range(L):
                    cap[i] = cap[i] + row(_B_H + (j * I + i) * L + l) * gs[l]

        x = jnp.zeros((16,), jnp.float32)
        for l in range(L):
            c = row(_B_B + l)
            for i in range(I):
                c = c + cap[i] * row(_B_W + l * I + i)
            x = x + c * c

        yi = 0x5F3759DF - (plsc.bitcast(x, jnp.int32) >> 1)
        y = plsc.bitcast(yi, jnp.float32)
        for _ in range(3):
            y = y * (1.5 - 0.5 * x * y * y)
        p = x * y

        e = jnp.exp(p - jnp.max(p))
        o_v[...] = e / jnp.sum(e)
        pltpu.sync_copy(o_v, out_hbm)


@jax.jit
def kernel(state, constants, gammas, head_W, tail_W, tail_b):
    s = state.reshape(-1, W, L)[0]
    sb = jnp.broadcast_to(s.reshape(W * L, 1), (W * L, M))
    gamT = gammas[:, 1:J + 1, :].transpose(1, 2, 0).reshape(J * L, M)
    temT = constants[:, :J, :].transpose(1, 2, 0).reshape(J * L, M)
    hwT = head_W.transpose(1, 2, 3, 0).reshape(J * I * L, M)
    twT = tail_W.transpose(1, 2, 0).reshape(L * I, M)
    tbT = tail_b.transpose(1, 0).reshape(L, M)
    a = jnp.concatenate([sb, gamT, temT, hwT, twT, tbT], axis=0)

    run = pl.kernel(
        _body,
        out_type=jax.ShapeDtypeStruct((M,), jnp.float32),
        mesh=plsc.VectorSubcoreMesh(core_axis_name="c", subcore_axis_name="s"),
        scratch_types=[
            pltpu.VMEM((_ROWS, M), jnp.float32),
            pltpu.VMEM((M,), jnp.float32),
        ],
        compiler_params=pltpu.CompilerParams(needs_layout_passes=False),
    )
    return run(a)


# speedup vs baseline: 1.0507x; 1.0507x over previous
import functools

import jax
import jax.numpy as jnp
from jax import lax
from jax.experimental import pallas as pl
from jax.experimental.pallas import tpu as pltpu
from jax.experimental.pallas import tpu_sc as plsc

M, J, I, L, W = 16, 2, 3, 2, 9

_B_G, _B_T, _B_H, _B_W, _B_B = W * L, W * L + J * L, W * L + 2 * J * L, \
    W * L + 2 * J * L + J * I * L, W * L + 2 * J * L + J * I * L + L * I
_ROWS = _B_B + L


def _body(a_hbm, out_hbm, a_v, o_v):
    @pl.when((lax.axis_index("c") == 0) & (lax.axis_index("s") == 0))
    def _():
        pltpu.sync_copy(a_hbm, a_v)

        def row(r):
            return a_v[r]

        cap = [jnp.zeros((16,), jnp.float32) for _ in range(I)]
        for j in range(J):
            gam = [row(_B_G + j * L + l) for l in range(L)]
            sig = [1.0 / (1.0 + jnp.exp(-10.0 * (g - 0.5))) for g in gam]
            tem = [row(_B_T + j * L + l) for l in range(L)]
            best_q = None
            sel = [None] * L
            for w in range(W):
                d0 = tem[0] - row(w * L + 0)
                q = sig[0] * d0 * d0
                for l in range(1, L):
                    dl = tem[l] - row(w * L + l)
                    q = q + sig[l] * dl * dl
                if best_q is None:
                    best_q = q
                    sel = [row(w * L + l) for l in range(L)]
                else:
                    take = q < best_q
                    best_q = jnp.where(take, q, best_q)
                    sel = [jnp.where(take, row(w * L + l), sel[l])
                           for l in range(L)]
            gs = [jnp.where(g > 0.5, sel[l], 0.0)
                  for l, g in enumerate(gam)]
            for i in range(I):
                for l in range(L):
                    cap[i] = cap[i] + row(_B_H + (j * I + i) * L + l) * gs[l]

        x = jnp.zeros((16,), jnp.float32)
        for l in range(L):
            c = row(_B_B + l)
            for i in range(I):
                c = c + cap[i] * row(_B_W + l * I + i)
            x = x + c * c

        yi = 0x5F3759DF - (plsc.bitcast(x, jnp.int32) >> 1)
        y = plsc.bitcast(yi, jnp.float32)
        for _ in range(3):
            y = y * (1.5 - 0.5 * x * y * y)
        p = x * y

        e = jnp.exp(p - jnp.max(p))
        o_v[...] = e / jnp.sum(e)
        pltpu.sync_copy(o_v, out_hbm)


@jax.jit
def kernel(state, constants, gammas, head_W, tail_W, tail_b):
    s = state.reshape(-1, W, L)[0]
    sb = jnp.broadcast_to(s.reshape(W * L, 1), (W * L, M))
    gamT = gammas[:, 1:J + 1, :].transpose(1, 2, 0).reshape(J * L, M)
    temT = constants[:, :J, :].transpose(1, 2, 0).reshape(J * L, M)
    hwT = head_W.transpose(1, 2, 3, 0).reshape(J * I * L, M)
    twT = tail_W.transpose(1, 2, 0).reshape(L * I, M)
    tbT = tail_b.transpose(1, 0).reshape(L, M)
    a = jnp.concatenate([sb, gamT, temT, hwT, twT, tbT], axis=0)

    run = pl.kernel(
        _body,
        out_type=jax.ShapeDtypeStruct((M,), jnp.float32),
        mesh=plsc.VectorSubcoreMesh(core_axis_name="c", subcore_axis_name="s",
                                    num_cores=1, num_subcores=1),
        scratch_types=[
            pltpu.VMEM((_ROWS, M), jnp.float32),
            pltpu.VMEM((M,), jnp.float32),
        ],
        compiler_params=pltpu.CompilerParams(needs_layout_passes=False),
    )
    return run(a)


# speedup vs baseline: 1.0671x; 1.0156x over previous
import functools

import jax
import jax.numpy as jnp
from jax import lax
from jax.experimental import pallas as pl
from jax.experimental.pallas import tpu as pltpu
from jax.experimental.pallas import tpu_sc as plsc

M, J, I, L, W = 16, 2, 3, 2, 9

_B_G, _B_T, _B_H, _B_W, _B_B = W * L, W * L + J * L, W * L + 2 * J * L, \
    W * L + 2 * J * L + J * I * L, W * L + 2 * J * L + J * I * L + L * I
_ROWS = _B_B + L


def _body(a_hbm, out_hbm, a_v, o_v):
    @pl.when((lax.axis_index("c") == 0) & (lax.axis_index("s") == 0))
    def _():
        pltpu.sync_copy(a_hbm, a_v)

        def row(r):
            return a_v[r]

        cap = [jnp.zeros((16,), jnp.float32) for _ in range(I)]
        for j in range(J):
            gam = [row(_B_G + j * L + l) for l in range(L)]
            sig = [1.0 / (1.0 + jnp.exp(-10.0 * (g - 0.5))) for g in gam]
            tem = [row(_B_T + j * L + l) for l in range(L)]
            best_q = None
            sel = [None] * L
            for w in range(W):
                d0 = tem[0] - row(w * L + 0)
                q = sig[0] * d0 * d0
                for l in range(1, L):
                    dl = tem[l] - row(w * L + l)
                    q = q + sig[l] * dl * dl
                if best_q is None:
                    best_q = q
                    sel = [row(w * L + l) for l in range(L)]
                else:
                    take = q < best_q
                    best_q = jnp.where(take, q, best_q)
                    sel = [jnp.where(take, row(w * L + l), sel[l])
                           for l in range(L)]
            gs = [jnp.where(g > 0.5, sel[l], 0.0)
                  for l, g in enumerate(gam)]
            for i in range(I):
                for l in range(L):
                    cap[i] = cap[i] + row(_B_H + (j * I + i) * L + l) * gs[l]

        x = jnp.zeros((16,), jnp.float32)
        for l in range(L):
            c = row(_B_B + l)
            for i in range(I):
                c = c + cap[i] * row(_B_W + l * I + i)
            x = x + c * c

        yi = 0x5F3759DF - (plsc.bitcast(x, jnp.int32) >> 1)
        y = plsc.bitcast(yi, jnp.float32)
        for _ in range(3):
            y = y * (1.5 - 0.5 * x * y * y)
        p = x * y

        e = jnp.exp(p - jnp.max(p))
        o_v[...] = e / jnp.sum(e)
        pltpu.sync_copy(o_v, out_hbm)


@jax.jit
def kernel(state, constants, gammas, head_W, tail_W, tail_b):
    s = state.reshape(-1, W, L)[0]
    sb = jnp.broadcast_to(s.reshape(W * L, 1), (W * L, M))
    gamT = gammas[:, 1:J + 1, :].transpose(1, 2, 0).reshape(J * L, M)
    temT = constants[:, :J, :].transpose(1, 2, 0).reshape(J * L, M)
    hwT = head_W.transpose(1, 2, 3, 0).reshape(J * I * L, M)
    twT = tail_W.transpose(1, 2, 0).reshape(L * I, M)
    tbT = tail_b.transpose(1, 0).reshape(L, M)
    a = jnp.concatenate([sb, gamT, temT, hwT, twT, tbT], axis=0)

    run = pl.kernel(
        _body,
        out_type=jax.ShapeDtypeStruct((M,), jnp.float32),
        mesh=plsc.VectorSubcoreMesh(core_axis_name="c", subcore_axis_name="s",
                                    num_cores=1, num_subcores=1),
        scratch_types=[
            pltpu.VMEM((_ROWS, M), jnp.float32),
            pltpu.VMEM((M,), jnp.float32),
        ],
        compiler_params=pltpu.CompilerParams(needs_layout_passes=False,
                                             skip_device_barrier=True),
    )
    return run(a)


# speedup vs baseline: 1.3397x; 1.2555x over previous
import jax
import jax.numpy as jnp
from jax import lax
from jax.experimental import pallas as pl
from jax.experimental.pallas import tpu as pltpu
from jax.experimental.pallas import tpu_sc as plsc

M, J, I, L, W = 16, 2, 3, 2, 9


def _iota():
    return lax.iota(jnp.int32, 16)


def _splat(c):
    return jnp.full((16,), c, jnp.int32)


def _body(st_hbm, con_hbm, gam_hbm, hw_hbm, tw_hbm, tb_hbm, out_hbm,
          st_v, con_v, gam_v, hw_v, tw_v, tb_v, o_v, sem):
    @pl.when((lax.axis_index("c") == 0) & (lax.axis_index("s") == 0))
    def _():
        cps = [pltpu.async_copy(src, dst, sem) for src, dst in
               [(st_hbm, st_v), (con_hbm, con_v), (gam_hbm, gam_v),
                (hw_hbm, hw_v), (tw_hbm, tw_v), (tb_hbm, tb_v)]]
        for cp in cps:
            cp.wait()

        ii = _iota()

        def splat_s(w, l):
            return plsc.load_gather(st_v, [_splat(w * L + l)])

        def vec(ref, stride, off):
            return plsc.load_gather(ref, [ii * stride + _splat(off)])

        s = [[splat_s(w, l) for l in range(L)] for w in range(W)]

        cap = [jnp.zeros((16,), jnp.float32) for _ in range(I)]
        for j in range(J):
            gam = [vec(gam_v, (J + 1) * L, (j + 1) * L + l) for l in range(L)]
            sig = [1.0 / (1.0 + jnp.exp(-10.0 * (g - 0.5))) for g in gam]
            tem = [vec(con_v, (J + 1) * L, j * L + l) for l in range(L)]
            best_q = None
            sel = [None] * L
            for w in range(W):
                d0 = tem[0] - s[w][0]
                q = sig[0] * d0 * d0
                for l in range(1, L):
                    dl = tem[l] - s[w][l]
                    q = q + sig[l] * dl * dl
                if best_q is None:
                    best_q = q
                    sel = list(s[w])
                else:
                    take = q < best_q
                    best_q = jnp.where(take, q, best_q)
                    sel = [jnp.where(take, s[w][l], sel[l]) for l in range(L)]
            gs = [jnp.where(g > 0.5, sel[l], 0.0) for l, g in enumerate(gam)]
            for i in range(I):
                for l in range(L):
                    cap[i] = cap[i] + vec(hw_v, J * I * L, (j * I + i) * L + l) * gs[l]

        x = jnp.zeros((16,), jnp.float32)
        for l in range(L):
            c = vec(tb_v, L, l)
            for i in range(I):
                c = c + cap[i] * vec(tw_v, L * I, l * I + i)
            x = x + c * c

        yi = 0x5F3759DF - (plsc.bitcast(x, jnp.int32) >> 1)
        y = plsc.bitcast(yi, jnp.float32)
        for _ in range(3):
            y = y * (1.5 - 0.5 * x * y * y)
        p = x * y

        e = jnp.exp(p - jnp.max(p))
        o_v[...] = e / jnp.sum(e)
        pltpu.sync_copy(o_v, out_hbm)


@jax.jit
def kernel(state, constants, gammas, head_W, tail_W, tail_b):
    run = pl.kernel(
        _body,
        out_type=jax.ShapeDtypeStruct((M,), jnp.float32),
        mesh=plsc.VectorSubcoreMesh(core_axis_name="c", subcore_axis_name="s",
                                    num_cores=1, num_subcores=1),
        scratch_types=[
            pltpu.VMEM((W * L,), jnp.float32),
            pltpu.VMEM((M * (J + 1) * L,), jnp.float32),
            pltpu.VMEM((M * (J + 1) * L,), jnp.float32),
            pltpu.VMEM((M * J * I * L,), jnp.float32),
            pltpu.VMEM((M * L * I,), jnp.float32),
            pltpu.VMEM((M * L,), jnp.float32),
            pltpu.VMEM((M,), jnp.float32),
            pltpu.SemaphoreType.DMA,
        ],
        compiler_params=pltpu.CompilerParams(needs_layout_passes=False,
                                             skip_device_barrier=True),
    )
    return run(state.ravel(), constants.ravel(), gammas.ravel(),
               head_W.ravel(), tail_W.ravel(), tail_b.ravel())
